# Initial kernel scaffold; baseline (speedup 1.0000x reference)
import jax
import jax.numpy as jnp
from jax.experimental import pallas as pl


def kernel(x, edge_index, Wl1, bl1, Wr1, Wl2, bl2, Wr2, Wl3, bl3, Wr3, Wc, bc):
    raise NotImplementedError("write your pallas kernel here")


# speedup vs baseline: 6.0405x; 6.0405x over previous
import functools

import jax
import jax.numpy as jnp
from jax import lax
from jax.experimental import pallas as pl
from jax.experimental.pallas import tpu as pltpu
from jax.experimental.pallas import tpu_sc as plsc

_N = 10000
_E = 320000
_F = 128
_NC = 2
_NS = 16
_NW = _NC * _NS
_EPT = _E // _NW
_CH = 80
_NCHK = _EPT // _CH
_NP = 10240
_RPT = _NP // _NS
_ZR = 128

_R = 1000
_G = _N // _R

_mesh = plsc.VectorSubcoreMesh(core_axis_name="c", subcore_axis_name="s")


def _zero_fill_2d(ref, nrows, ncols):
    z16 = jnp.zeros((16,), jnp.float32)

    def _row(r, carry):
        for k in range(ncols // 16):
            ref[r, pl.ds(k * 16, 16)] = z16
        return carry

    lax.fori_loop(0, nrows, _row, 0)


@functools.partial(
    pl.kernel,
    mesh=_mesh,
    out_type=[
        jax.ShapeDtypeStruct((_NC, _NP, _F), jnp.float32),
        jax.ShapeDtypeStruct((_NC, _NP, 16), jnp.float32),
    ],
    scratch_types=[
        pltpu.VMEM_SHARED((_NP, _F), jnp.float32),
        pltpu.VMEM_SHARED((_NP, 16), jnp.float32),
        pltpu.VMEM((_CH,), jnp.int32),
        pltpu.VMEM((_CH,), jnp.int32),
        pltpu.VMEM((_CH, _F), jnp.float32),
        pltpu.VMEM((_CH, 16), jnp.float32),
        pltpu.VMEM((_ZR, _F), jnp.float32),
        pltpu.VMEM((_RPT, 16), jnp.float32),
        pltpu.SemaphoreType.DMA,
    ],
    compiler_params=pltpu.CompilerParams(use_tc_tiling_on_sc=False),
)
def _sc_segsum_counts(src_hbm, dst_hbm, x_hbm, p_out, cnt_out,
                      acc_sh, cnt_sh, sidx, didx, rows, ones16, zbuf,
                      zbuf16, sem):
    c = lax.axis_index("c")
    s = lax.axis_index("s")
    wid = c * _NS + s

    _zero_fill_2d(zbuf, _ZR, _F)
    _zero_fill_2d(zbuf16, _RPT, 16)
    o16 = jnp.ones((16,), jnp.float32)

    def _ones_row(r, carry):
        ones16[r, :] = o16
        return carry

    lax.fori_loop(0, _CH, _ones_row, 0)

    r0 = s * _RPT
    for k in range(_RPT // _ZR):
        pltpu.sync_copy(zbuf, acc_sh.at[pl.ds(r0 + k * _ZR, _ZR)])
    pltpu.sync_copy(zbuf16, cnt_sh.at[pl.ds(r0, _RPT)])
    plsc.subcore_barrier()

    def _chunk(j, carry):
        pltpu.sync_copy(src_hbm.at[wid, j], sidx)
        pltpu.sync_copy(dst_hbm.at[wid, j], didx)
        pltpu.async_copy(x_hbm.at[sidx], rows, sem).wait()
        pltpu.sync_copy(rows, acc_sh.at[didx], add=True)
        pltpu.sync_copy(ones16, cnt_sh.at[didx], add=True)
        return carry

    lax.fori_loop(0, _NCHK, _chunk, 0)
    plsc.subcore_barrier()

    pltpu.sync_copy(acc_sh.at[pl.ds(r0, _RPT)], p_out.at[c, pl.ds(r0, _RPT)])
    pltpu.sync_copy(cnt_sh.at[pl.ds(r0, _RPT)], cnt_out.at[c, pl.ds(r0, _RPT)])


@functools.partial(
    pl.kernel,
    mesh=_mesh,
    out_type=[
        jax.ShapeDtypeStruct((_NC, _NP, _F), jnp.float32),
        jax.ShapeDtypeStruct((_NC, _NP, 16), jnp.float32),
    ],
    scratch_types=[
        pltpu.VMEM_SHARED((_NP, _F), jnp.float32),
        pltpu.VMEM_SHARED((_NP, 16), jnp.float32),
        pltpu.VMEM((_CH,), jnp.int32),
        pltpu.VMEM((_CH,), jnp.int32),
        pltpu.VMEM((_CH, _F), jnp.float32),
        pltpu.VMEM((_CH, 16), jnp.float32),
        pltpu.VMEM((_ZR, _F), jnp.float32),
        pltpu.VMEM((_RPT, 16), jnp.float32),
        pltpu.SemaphoreType.DMA,
    ],
    compiler_params=pltpu.CompilerParams(use_tc_tiling_on_sc=False),
)
def _sc_segsum_cvec(src_hbm, dst_hbm, h_hbm, inv16_hbm, q_out, cvec_out,
                    acc_sh, cvec_sh, sidx, didx, rows, w16, zbuf, zbuf16,
                    sem):
    c = lax.axis_index("c")
    s = lax.axis_index("s")
    wid = c * _NS + s

    _zero_fill_2d(zbuf, _ZR, _F)
    _zero_fill_2d(zbuf16, _RPT, 16)

    r0 = s * _RPT
    for k in range(_RPT // _ZR):
        pltpu.sync_copy(zbuf, acc_sh.at[pl.ds(r0 + k * _ZR, _ZR)])
    pltpu.sync_copy(zbuf16, cvec_sh.at[pl.ds(r0, _RPT)])
    plsc.subcore_barrier()

    def _chunk(j, carry):
        pltpu.sync_copy(src_hbm.at[wid, j], sidx)
        pltpu.sync_copy(dst_hbm.at[wid, j], didx)
        pltpu.async_copy(h_hbm.at[sidx], rows, sem).wait()
        pltpu.sync_copy(rows, acc_sh.at[didx], add=True)
        pltpu.async_copy(inv16_hbm.at[didx], w16, sem).wait()
        pltpu.sync_copy(w16, cvec_sh.at[sidx], add=True)
        return carry

    lax.fori_loop(0, _NCHK, _chunk, 0)
    plsc.subcore_barrier()

    pltpu.sync_copy(acc_sh.at[pl.ds(r0, _RPT)], q_out.at[c, pl.ds(r0, _RPT)])
    pltpu.sync_copy(cvec_sh.at[pl.ds(r0, _RPT)], cvec_out.at[c, pl.ds(r0, _RPT)])


def _dotT(a, b):
    return lax.dot_general(a, b, (((1,), (1,)), ((), ())),
                           preferred_element_type=jnp.float32)


def _dense1_body(p_ref, cnt_ref, x_ref, wl_ref, bl_ref, wr_ref,
                 h_ref, inv16_ref):
    cnt = cnt_ref[0][:, 0:1] + cnt_ref[1][:, 0:1]
    inv = 1.0 / jnp.maximum(cnt, 1.0)
    agg = (p_ref[0] + p_ref[1]) * inv
    h = _dotT(agg, wl_ref[...]) + _dotT(x_ref[...], wr_ref[...]) + bl_ref[...]
    h_ref[...] = jnp.maximum(h, 0.0)
    inv16_ref[...] = jnp.broadcast_to(inv, (_R, 16))


def _dense2_body(q_ref, cv_ref, h1_ref, inv16_ref, wl2_ref, bl2_ref, wr2_ref,
                 wl3_ref, bl3_ref, wr3_ref, wc_ref, bc_ref,
                 out_ref, g1_acc, g2_acc):
    i = pl.program_id(0)
    inv = inv16_ref[:, 0:1]
    agg = (q_ref[0] + q_ref[1]) * inv
    h2 = _dotT(agg, wl2_ref[...]) + _dotT(h1_ref[...], wr2_ref[...]) + bl2_ref[...]
    h2 = jnp.maximum(h2, 0.0)
    cv = cv_ref[0][:, 0:1] + cv_ref[1][:, 0:1]
    part1 = jnp.sum(cv * h2, axis=0, keepdims=True)
    part2 = jnp.sum(h2, axis=0, keepdims=True)

    @pl.when(i == 0)
    def _():
        g1_acc[...] = jnp.zeros_like(g1_acc)
        g2_acc[...] = jnp.zeros_like(g2_acc)

    g1_acc[...] += part1
    g2_acc[...] += part2

    @pl.when(i == _G - 1)
    def _():
        g1 = g1_acc[...] * (1.0 / _N)
        g2 = g2_acc[...] * (1.0 / _N)
        gm = _dotT(g1, wl3_ref[...]) + bl3_ref[...] + _dotT(g2, wr3_ref[...])
        out_ref[...] = _dotT(gm, wc_ref[...]) + bc_ref[...]


def kernel(x, edge_index, Wl1, bl1, Wr1, Wl2, bl2, Wr2, Wl3, bl3, Wr3, Wc, bc):
    src_r = edge_index[0].reshape(_NW, _NCHK, _CH)
    dst_r = edge_index[1].reshape(_NW, _NCHK, _CH)

    p1, cnt16 = _sc_segsum_counts(src_r, dst_r, x)

    wfull = pl.BlockSpec((_F, _F), lambda i: (0, 0))
    bfull = pl.BlockSpec((1, _F), lambda i: (0, 0))
    rowblk = pl.BlockSpec((_R, _F), lambda i: (i, 0))
    pblk = pl.BlockSpec((_NC, _R, _F), lambda i: (0, i, 0))
    cblk = pl.BlockSpec((_NC, _R, 16), lambda i: (0, i, 0))
    s16blk = pl.BlockSpec((_R, 16), lambda i: (i, 0))

    h1, inv16 = pl.pallas_call(
        _dense1_body,
        grid=(_G,),
        in_specs=[pblk, cblk, rowblk, wfull, bfull, wfull],
        out_specs=[rowblk, s16blk],
        out_shape=[jax.ShapeDtypeStruct((_N, _F), jnp.float32),
                   jax.ShapeDtypeStruct((_N, 16), jnp.float32)],
    )(p1, cnt16, x, Wl1, bl1.reshape(1, _F), Wr1)

    q2, cvec16 = _sc_segsum_cvec(src_r, dst_r, h1, inv16)

    out = pl.pallas_call(
        _dense2_body,
        grid=(_G,),
        in_specs=[pblk, cblk, rowblk, s16blk, wfull, bfull, wfull,
                  wfull, bfull, wfull,
                  pl.BlockSpec((Wc.shape[0], _F), lambda i: (0, 0)),
                  pl.BlockSpec((1, Wc.shape[0]), lambda i: (0, 0))],
        out_specs=pl.BlockSpec((1, Wc.shape[0]), lambda i: (0, 0)),
        out_shape=jax.ShapeDtypeStruct((1, Wc.shape[0]), jnp.float32),
        scratch_shapes=[pltpu.VMEM((1, _F), jnp.float32),
                        pltpu.VMEM((1, _F), jnp.float32)],
    )(q2, cvec16, h1, inv16, Wl2, bl2.reshape(1, _F), Wr2,
      Wl3, bl3.reshape(1, _F), Wr3, Wc, bc.reshape(1, -1))

    return out


# speedup vs baseline: 10.5777x; 1.7511x over previous
import functools

import jax
import jax.numpy as jnp
from jax import lax
from jax.experimental import pallas as pl
from jax.experimental.pallas import tpu as pltpu
from jax.experimental.pallas import tpu_sc as plsc

_N = 10000
_E = 320000
_F = 128
_NC = 2
_NS = 16
_NW = _NC * _NS
_EPT = _E // _NW
_CH = 80
_NCHK = _EPT // _CH
_NP = 10240
_RPT = _NP // _NS
_ZR = 40

_R = 1000
_G = _N // _R

_mesh = plsc.VectorSubcoreMesh(core_axis_name="c", subcore_axis_name="s")


def _zero_fill_2d(ref, nrows, ncols):
    z16 = jnp.zeros((16,), jnp.float32)

    def _row(r, carry):
        for k in range(ncols // 16):
            ref[r, pl.ds(k * 16, 16)] = z16
        return carry

    lax.fori_loop(0, nrows, _row, 0)


@functools.partial(
    pl.kernel,
    mesh=_mesh,
    out_type=[
        jax.ShapeDtypeStruct((_NC, _NP, _F), jnp.float32),
        jax.ShapeDtypeStruct((_NC, _NP, 16), jnp.float32),
    ],
    scratch_types=[
        pltpu.VMEM_SHARED((_NP, _F), jnp.float32),
        pltpu.VMEM_SHARED((_NP, 16), jnp.float32),
        pltpu.VMEM((2, _CH), jnp.int32),
        pltpu.VMEM((2, _CH), jnp.int32),
        pltpu.VMEM((2, _CH, _F), jnp.float32),
        pltpu.VMEM((_CH, 16), jnp.float32),
        pltpu.VMEM((_ZR, _F), jnp.float32),
        pltpu.VMEM((_ZR, 16), jnp.float32),
        pltpu.SemaphoreType.DMA,
        pltpu.SemaphoreType.DMA,
    ],
    compiler_params=pltpu.CompilerParams(use_tc_tiling_on_sc=False,
                                         needs_layout_passes=False),
)
def _sc_segsum_counts(src_hbm, dst_hbm, x_hbm, p_out, cnt_out,
                      acc_sh, cnt_sh, sidx, didx, rows, ones16, zbuf,
                      zbuf16, sem0, sem1):
    c = lax.axis_index("c")
    s = lax.axis_index("s")
    wid = c * _NS + s
    r0 = s * _RPT

    pltpu.sync_copy(src_hbm.at[wid, 0], sidx.at[0])
    pltpu.sync_copy(dst_hbm.at[wid, 0], didx.at[0])
    pltpu.async_copy(x_hbm.at[sidx.at[0]], rows.at[0], sem0)
    pltpu.sync_copy(src_hbm.at[wid, 1], sidx.at[1])
    pltpu.sync_copy(dst_hbm.at[wid, 1], didx.at[1])
    pltpu.async_copy(x_hbm.at[sidx.at[1]], rows.at[1], sem1)

    _zero_fill_2d(zbuf, _ZR, _F)
    _zero_fill_2d(zbuf16, _ZR, 16)
    o16 = jnp.ones((16,), jnp.float32)

    def _ones_row(r, carry):
        ones16[r, :] = o16
        return carry

    lax.fori_loop(0, _CH, _ones_row, 0)

    for k in range(_RPT // _ZR):
        pltpu.sync_copy(zbuf, acc_sh.at[pl.ds(r0 + k * _ZR, _ZR)])
        pltpu.sync_copy(zbuf16, cnt_sh.at[pl.ds(r0 + k * _ZR, _ZR)])
    plsc.subcore_barrier()

    def _pair(i, carry):
        j0 = 2 * i
        pltpu.make_async_copy(x_hbm.at[sidx.at[0]], rows.at[0], sem0).wait()
        pltpu.sync_copy(rows.at[0], acc_sh.at[didx.at[0]], add=True)
        pltpu.sync_copy(ones16, cnt_sh.at[didx.at[0]], add=True)
        pltpu.sync_copy(src_hbm.at[wid, j0 + 2], sidx.at[0])
        pltpu.sync_copy(dst_hbm.at[wid, j0 + 2], didx.at[0])
        pltpu.async_copy(x_hbm.at[sidx.at[0]], rows.at[0], sem0)
        pltpu.make_async_copy(x_hbm.at[sidx.at[1]], rows.at[1], sem1).wait()
        pltpu.sync_copy(rows.at[1], acc_sh.at[didx.at[1]], add=True)
        pltpu.sync_copy(ones16, cnt_sh.at[didx.at[1]], add=True)

        @pl.when(j0 + 3 < _NCHK)
        def _():
            pltpu.sync_copy(src_hbm.at[wid, j0 + 3], sidx.at[1])
            pltpu.sync_copy(dst_hbm.at[wid, j0 + 3], didx.at[1])
            pltpu.async_copy(x_hbm.at[sidx.at[1]], rows.at[1], sem1)

        return carry

    lax.fori_loop(0, (_NCHK - 1) // 2, _pair, 0)
    pltpu.make_async_copy(x_hbm.at[sidx.at[0]], rows.at[0], sem0).wait()
    pltpu.sync_copy(rows.at[0], acc_sh.at[didx.at[0]], add=True)
    pltpu.sync_copy(ones16, cnt_sh.at[didx.at[0]], add=True)
    plsc.subcore_barrier()

    pltpu.sync_copy(acc_sh.at[pl.ds(r0, _RPT)], p_out.at[c, pl.ds(r0, _RPT)])
    pltpu.sync_copy(cnt_sh.at[pl.ds(r0, _RPT)], cnt_out.at[c, pl.ds(r0, _RPT)])


@functools.partial(
    pl.kernel,
    mesh=_mesh,
    out_type=[
        jax.ShapeDtypeStruct((_NC, _NP, _F), jnp.float32),
        jax.ShapeDtypeStruct((_NC, _NP), jnp.float32),
    ],
    scratch_types=[
        pltpu.VMEM_SHARED((_NP, _F), jnp.float32),
        pltpu.VMEM_SHARED((_NP,), jnp.float32),
        pltpu.VMEM_SHARED((_NP,), jnp.float32),
        pltpu.VMEM((2, _CH), jnp.int32),
        pltpu.VMEM((2, _CH), jnp.int32),
        pltpu.VMEM((2, _CH, _F), jnp.float32),
        pltpu.VMEM((_CH,), jnp.float32),
        pltpu.VMEM((_ZR, _F), jnp.float32),
        pltpu.VMEM((_RPT,), jnp.float32),
        pltpu.VMEM((160, 16), jnp.float32),
        pltpu.VMEM((160, 16), jnp.float32),
        pltpu.VMEM((_NP,), jnp.float32),
        pltpu.SemaphoreType.DMA,
        pltpu.SemaphoreType.DMA,
    ],
    compiler_params=pltpu.CompilerParams(use_tc_tiling_on_sc=False,
                                         needs_layout_passes=False),
)
def _sc_segsum_cvec(src_hbm, dst_hbm, h_hbm, cnt_hbm, q_out, cvec_out,
                    acc_sh, cvec_sh, inv_sh, sidx, didx, rows, w, zbuf,
                    tmp1, cbuf0, cbuf1, invloc, sem0, sem1):
    c = lax.axis_index("c")
    s = lax.axis_index("s")
    wid = c * _NS + s
    r0 = s * _RPT

    pltpu.sync_copy(src_hbm.at[wid, 0], sidx.at[0])
    pltpu.sync_copy(dst_hbm.at[wid, 0], didx.at[0])
    pltpu.async_copy(h_hbm.at[sidx.at[0]], rows.at[0], sem0)
    pltpu.sync_copy(src_hbm.at[wid, 1], sidx.at[1])
    pltpu.sync_copy(dst_hbm.at[wid, 1], didx.at[1])
    pltpu.async_copy(h_hbm.at[sidx.at[1]], rows.at[1], sem1)

    zidx = jnp.zeros((16,), jnp.int32)
    i16 = lax.iota(jnp.int32, 16)
    for q in range(4):
        pltpu.sync_copy(cnt_hbm.at[0, pl.ds(r0 + q * 160, 160)], cbuf0)
        pltpu.sync_copy(cnt_hbm.at[1, pl.ds(r0 + q * 160, 160)], cbuf1)

        def _inv_grp(g, carry):
            ridx = i16 + g * 16
            c0 = plsc.load_gather(cbuf0, [ridx, zidx])
            c1 = plsc.load_gather(cbuf1, [ridx, zidx])
            tmp1[pl.ds(q * 160 + g * 16, 16)] = 1.0 / jnp.maximum(c0 + c1, 1.0)
            return carry

        lax.fori_loop(0, 10, _inv_grp, 0)
    pltpu.sync_copy(tmp1, inv_sh.at[pl.ds(r0, _RPT)])

    _zero_fill_2d(zbuf, _ZR, _F)
    for k in range(_RPT // _ZR):
        pltpu.sync_copy(zbuf, acc_sh.at[pl.ds(r0 + k * _ZR, _ZR)])

    def _z1(i, carry):
        tmp1[pl.ds(i * 16, 16)] = jnp.zeros((16,), jnp.float32)
        return carry

    lax.fori_loop(0, _RPT // 16, _z1, 0)
    pltpu.sync_copy(tmp1, cvec_sh.at[pl.ds(r0, _RPT)])
    plsc.subcore_barrier()
    pltpu.sync_copy(inv_sh, invloc)

    def _wvals(b):
        for k in range(_CH // 16):
            d16 = didx[b, pl.ds(k * 16, 16)]
            w[pl.ds(k * 16, 16)] = plsc.load_gather(invloc, [d16])

    def _pair(i, carry):
        j0 = 2 * i
        pltpu.make_async_copy(h_hbm.at[sidx.at[0]], rows.at[0], sem0).wait()
        pltpu.sync_copy(rows.at[0], acc_sh.at[didx.at[0]], add=True)
        _wvals(0)
        pltpu.sync_copy(w, cvec_sh.at[sidx.at[0]], add=True)
        pltpu.sync_copy(src_hbm.at[wid, j0 + 2], sidx.at[0])
        pltpu.sync_copy(dst_hbm.at[wid, j0 + 2], didx.at[0])
        pltpu.async_copy(h_hbm.at[sidx.at[0]], rows.at[0], sem0)
        pltpu.make_async_copy(h_hbm.at[sidx.at[1]], rows.at[1], sem1).wait()
        pltpu.sync_copy(rows.at[1], acc_sh.at[didx.at[1]], add=True)
        _wvals(1)
        pltpu.sync_copy(w, cvec_sh.at[sidx.at[1]], add=True)

        @pl.when(j0 + 3 < _NCHK)
        def _():
            pltpu.sync_copy(src_hbm.at[wid, j0 + 3], sidx.at[1])
            pltpu.sync_copy(dst_hbm.at[wid, j0 + 3], didx.at[1])
            pltpu.async_copy(h_hbm.at[sidx.at[1]], rows.at[1], sem1)

        return carry

    lax.fori_loop(0, (_NCHK - 1) // 2, _pair, 0)
    pltpu.make_async_copy(h_hbm.at[sidx.at[0]], rows.at[0], sem0).wait()
    pltpu.sync_copy(rows.at[0], acc_sh.at[didx.at[0]], add=True)
    _wvals(0)
    pltpu.sync_copy(w, cvec_sh.at[sidx.at[0]], add=True)
    plsc.subcore_barrier()

    pltpu.sync_copy(acc_sh.at[pl.ds(r0, _RPT)], q_out.at[c, pl.ds(r0, _RPT)])
    pltpu.sync_copy(cvec_sh.at[pl.ds(r0, _RPT)], cvec_out.at[c, pl.ds(r0, _RPT)])


def _dotT(a, b):
    return lax.dot_general(a, b, (((1,), (1,)), ((), ())),
                           preferred_element_type=jnp.float32)


def _dense1_body(p_ref, cnt_ref, x_ref, wl_ref, bl_ref, wr_ref, h_ref):
    cnt = cnt_ref[0][:, 0:1] + cnt_ref[1][:, 0:1]
    inv = 1.0 / jnp.maximum(cnt, 1.0)
    agg = (p_ref[0] + p_ref[1]) * inv
    h = _dotT(agg, wl_ref[...]) + _dotT(x_ref[...], wr_ref[...]) + bl_ref[...]
    h_ref[...] = jnp.maximum(h, 0.0)


def _dense2_body(q_ref, cv_ref, h1_ref, cnt_ref, wl2_ref, bl2_ref, wr2_ref,
                 wl3_ref, bl3_ref, wr3_ref, wc_ref, bc_ref,
                 out_ref, g1_acc, g2_acc):
    i = pl.program_id(0)
    cnt = cnt_ref[0][:, 0:1] + cnt_ref[1][:, 0:1]
    inv = 1.0 / jnp.maximum(cnt, 1.0)
    agg = (q_ref[0] + q_ref[1]) * inv
    h2 = _dotT(agg, wl2_ref[...]) + _dotT(h1_ref[...], wr2_ref[...]) + bl2_ref[...]
    h2 = jnp.maximum(h2, 0.0)
    cv = cv_ref[:, 0:1] + cv_ref[:, 1:2]
    part1 = jnp.sum(cv * h2, axis=0, keepdims=True)
    part2 = jnp.sum(h2, axis=0, keepdims=True)

    @pl.when(i == 0)
    def _():
        g1_acc[...] = jnp.zeros_like(g1_acc)
        g2_acc[...] = jnp.zeros_like(g2_acc)

    g1_acc[...] += part1
    g2_acc[...] += part2

    @pl.when(i == _G - 1)
    def _():
        g1 = g1_acc[...] * (1.0 / _N)
        g2 = g2_acc[...] * (1.0 / _N)
        gm = _dotT(g1, wl3_ref[...]) + bl3_ref[...] + _dotT(g2, wr3_ref[...])
        out_ref[...] = _dotT(gm, wc_ref[...]) + bc_ref[...]


def kernel(x, edge_index, Wl1, bl1, Wr1, Wl2, bl2, Wr2, Wl3, bl3, Wr3, Wc, bc):
    src_r = edge_index[0].reshape(_NW, _NCHK, _CH)
    dst_r = edge_index[1].reshape(_NW, _NCHK, _CH)

    p1, cnt16 = _sc_segsum_counts(src_r, dst_r, x)

    wfull = pl.BlockSpec((_F, _F), lambda i: (0, 0))
    bfull = pl.BlockSpec((1, _F), lambda i: (0, 0))
    rowblk = pl.BlockSpec((_R, _F), lambda i: (i, 0))
    pblk = pl.BlockSpec((_NC, _R, _F), lambda i: (0, i, 0))
    cblk = pl.BlockSpec((_NC, _R, 16), lambda i: (0, i, 0))

    h1 = pl.pallas_call(
        _dense1_body,
        grid=(_G,),
        in_specs=[pblk, cblk, rowblk, wfull, bfull, wfull],
        out_specs=rowblk,
        out_shape=jax.ShapeDtypeStruct((_N, _F), jnp.float32),
    )(p1, cnt16, x, Wl1, bl1.reshape(1, _F), Wr1)

    q2, cvec = _sc_segsum_cvec(src_r, dst_r, h1, cnt16)

    cvblk = pl.BlockSpec((_R, _NC), lambda i: (i, 0))
    out = pl.pallas_call(
        _dense2_body,
        grid=(_G,),
        in_specs=[pblk, cvblk, rowblk, cblk, wfull, bfull, wfull,
                  wfull, bfull, wfull,
                  pl.BlockSpec((Wc.shape[0], _F), lambda i: (0, 0)),
                  pl.BlockSpec((1, Wc.shape[0]), lambda i: (0, 0))],
        out_specs=pl.BlockSpec((1, Wc.shape[0]), lambda i: (0, 0)),
        out_shape=jax.ShapeDtypeStruct((1, Wc.shape[0]), jnp.float32),
        scratch_shapes=[pltpu.VMEM((1, _F), jnp.float32),
                        pltpu.VMEM((1, _F), jnp.float32)],
    )(q2, cvec.T, h1, cnt16, Wl2, bl2.reshape(1, _F), Wr2,
      Wl3, bl3.reshape(1, _F), Wr3, Wc, bc.reshape(1, -1))

    return out
